# baseline (speedup 1.0000x reference)
import functools

import jax
import jax.numpy as jnp
from jax import lax
from jax.experimental import pallas as pl
from jax.experimental.pallas import tpu as pltpu
from jax.experimental.pallas import tpu_sc as plsc

B, T, D, E, TOP_K = 2, 2048, 1024, 8, 2
ROWS = B * T
NENT = TOP_K * ROWS
MT = 256
NPAD = NENT + E * MT
NT = NPAD // MT
GTILE = 1024

NW = 32
EPW = NENT // 16
DPW = NPAD // NW
GCH = 80
TPW = ROWS // NW
CCH = 32

_mesh = plsc.VectorSubcoreMesh(core_axis_name="c", subcore_axis_name="s")


def _gate_kernel(x_ref, wg_ref, gate_ref, xb_ref, a1_ref, a2_ref,
                 w1_ref, w2_ref):
    x = x_ref[...]
    xb = x.astype(jnp.bfloat16)

    wg = wg_ref[...].astype(jnp.bfloat16)
    logits = lax.dot_general(
        xb, wg, (((1,), (1,)), ((), ())),
        preferred_element_type=jnp.float32)

    m = jnp.max(logits, axis=1, keepdims=True)
    eg = jnp.exp(logits - m)
    gate = eg / jnp.sum(eg, axis=1, keepdims=True)

    col = lax.broadcasted_iota(jnp.int32, (GTILE, E), 1)
    a1 = jnp.argmax(gate, axis=1)[:, None]
    w1 = jnp.max(gate, axis=1)[:, None]
    masked = jnp.where(col == a1, -jnp.inf, gate)
    a2 = jnp.argmax(masked, axis=1)[:, None]
    w2 = jnp.max(masked, axis=1)[:, None]

    gate_ref[...] = gate
    xb_ref[...] = xb
    a1_ref[...] = a1.astype(jnp.float32)
    a2_ref[...] = a2.astype(jnp.float32)
    w1_ref[...] = w1
    w2_ref[...] = w2


@functools.partial(
    pl.kernel, mesh=_mesh,
    compiler_params=pltpu.CompilerParams(needs_layout_passes=False),
    out_type=[
        jax.ShapeDtypeStruct((NPAD,), jnp.float32),
        jax.ShapeDtypeStruct((NENT,), jnp.int32),
        jax.ShapeDtypeStruct((16,), jnp.int32),
        jax.ShapeDtypeStruct((NPAD, D // 2), jnp.int32),
    ],
    scratch_types=[
        pltpu.VMEM((EPW,), jnp.float32),
        pltpu.VMEM((EPW,), jnp.float32),
        pltpu.VMEM((4, 128), jnp.int32),
        pltpu.VMEM((EPW,), jnp.int32),
        pltpu.VMEM((4, 128), jnp.int32),
        pltpu.VMEM((16,), jnp.int32),
        pltpu.VMEM((16, 16), jnp.int32),
        pltpu.VMEM_SHARED((16, 16), jnp.int32),
        pltpu.VMEM_SHARED((NPAD,), jnp.int32),
        pltpu.VMEM_SHARED((NPAD,), jnp.float32),
        pltpu.VMEM((DPW,), jnp.float32),
        pltpu.VMEM((GCH,), jnp.int32),
        pltpu.VMEM((GCH, D // 2), jnp.int32),
        pltpu.SemaphoreType.DMA,
    ],
)
def _dispatch_kernel(ar_hbm, wc_hbm, xb_hbm, wr_hbm, pp_hbm, cnt_hbm,
                     xs_hbm, arv, wvm, pdv2, pv, tokd, cvm, allc, shared,
                     sh_sod, sh_wr, wbuf, idxv, rows, sem):
    sid = lax.axis_index("s")
    cid = lax.axis_index("c")
    r0 = sid * EPW
    tb = lax.rem(r0, ROWS)
    pltpu.sync_copy(ar_hbm.at[pl.ds(r0, EPW)], arv)
    pltpu.sync_copy(wc_hbm.at[pl.ds(r0, EPW)], wvm)
    lane = lax.iota(jnp.int32, 16)

    def splat(v):
        return jnp.full((16,), v, jnp.int32)

    zero16 = jnp.zeros((16,), jnp.int32)
    one16 = jnp.full((16,), 1, jnp.int32)

    def sumsplat(v):
        c = plsc.cumsum(v)
        r = lax.rev(plsc.cumsum(lax.rev(v, (0,))), (0,))
        return c + r - v

    def p1_body(k, run):
        idk = arv[pl.ds(k * 16, 16)].astype(jnp.int32)
        for e in range(E):
            mi = jnp.where(idk == splat(e), one16, zero16)
            run = run + jnp.where(lane == splat(e), sumsplat(mi), zero16)
        return run

    run = lax.fori_loop(0, EPW // 16, p1_body, zero16)

    cvm[...] = run
    pltpu.sync_copy(cvm, shared.at[sid])
    plsc.subcore_barrier()
    pltpu.sync_copy(shared, allc)

    tot = zero16
    myoff = zero16
    for w in range(16):
        cw = allc[w]
        tot = tot + cw
        sel = jnp.where(splat(w) < splat(sid), one16, zero16)
        myoff = myoff + cw * sel

    padded = lax.shift_left(
        lax.shift_right_logical(tot + splat(MT - 1), splat(8)), splat(8))
    offv = plsc.cumsum(padded) - padded
    base = offv + myoff

    npad16 = splat(NPAD - 1)

    def p2_body(k, run2):
        idk = arv[pl.ds(k * 16, 16)].astype(jnp.int32)
        dv = zero16
        for e in range(E):
            mi = jnp.where(idk == splat(e), one16, zero16)
            incl = plsc.cumsum(mi)
            bev = sumsplat(jnp.where(lane == splat(e), base + run2, zero16))
            dv = dv + mi * (bev + incl - one16)
            run2 = run2 + jnp.where(lane == splat(e), sumsplat(mi), zero16)
        dv = jnp.minimum(jnp.maximum(dv, zero16), npad16)
        pv[pl.ds(k * 16, 16)] = dv
        row = lax.div(k, 8)
        coff = lax.rem(k, 8) * 16
        pdv2[row, pl.ds(coff, 16)] = dv
        tokd[row, pl.ds(coff, 16)] = splat(tb + k * 16) + lane
        return run2

    lax.fori_loop(0, EPW // 16, p2_body, zero16)

    @pl.when(cid == 0)
    def _():
        pltpu.sync_copy(pv, pp_hbm.at[pl.ds(r0, EPW)])

    for k in range(EPW // 128):
        pltpu.sync_copy(tokd.at[k], sh_sod.at[pdv2.at[k]])
        pltpu.sync_copy(wvm.at[pl.ds(k * 128, 128)], sh_wr.at[pdv2.at[k]])

    @pl.when(jnp.logical_and(sid == 0, cid == 0))
    def _():
        cvm[...] = tot
        pltpu.sync_copy(cvm, cnt_hbm)

    plsc.subcore_barrier()

    dbase = (cid * 16 + sid) * DPW
    pltpu.sync_copy(sh_wr.at[pl.ds(dbase, DPW)], wbuf)
    pltpu.sync_copy(wbuf, wr_hbm.at[pl.ds(dbase, DPW)])
    for c in range(DPW // GCH):
        pltpu.sync_copy(sh_sod.at[pl.ds(dbase + c * GCH, GCH)], idxv)
        for k in range(GCH // 16):
            v = idxv[pl.ds(k * 16, 16)]
            idxv[pl.ds(k * 16, 16)] = jnp.clip(v, 0, ROWS - 1)
        pltpu.async_copy(xb_hbm.at[idxv], rows, sem).wait()
        pltpu.sync_copy(rows, xs_hbm.at[pl.ds(dbase + c * GCH, GCH)])


def _mm_kernel(s_ref, xs_ref, we_ref, be_ref, wr_ref, ys_ref):
    del s_ref
    y = lax.dot_general(
        xs_ref[...], we_ref[0], (((1,), (1,)), ((), ())),
        preferred_element_type=jnp.float32)
    y = (y + be_ref[0]) * wr_ref[0]
    ys_ref[...] = y.astype(jnp.bfloat16)


@functools.partial(
    pl.kernel, mesh=_mesh,
    compiler_params=pltpu.CompilerParams(needs_layout_passes=False),
    out_type=jax.ShapeDtypeStruct((ROWS, D // 2), jnp.int32),
    scratch_types=[
        pltpu.VMEM((CCH,), jnp.int32),
        pltpu.VMEM((CCH,), jnp.int32),
        pltpu.VMEM((CCH, D // 2), jnp.int32),
        pltpu.VMEM((CCH, D // 2), jnp.int32),
        pltpu.VMEM((CCH, D // 2), jnp.int32),
        pltpu.SemaphoreType.DMA,
        pltpu.SemaphoreType.DMA,
    ],
)
def _combine_kernel(ys_hbm, pp_hbm, ob_hbm, p1v, p2v, b1, b2, ob,
                    sem1, sem2):
    wid = lax.axis_index("s") * 2 + lax.axis_index("c")
    tbase = wid * TPW
    for c in range(TPW // CCH):
        pltpu.sync_copy(pp_hbm.at[pl.ds(tbase + c * CCH, CCH)], p1v)
        pltpu.sync_copy(pp_hbm.at[pl.ds(ROWS + tbase + c * CCH, CCH)], p2v)
        cp1 = pltpu.async_copy(ys_hbm.at[p1v], b1, sem1)
        cp2 = pltpu.async_copy(ys_hbm.at[p2v], b2, sem2)
        cp1.wait()
        cp2.wait()

        zb = jnp.zeros((32,), jnp.bfloat16)

        def body(r, carry):
            for g in range(D // 32):
                va = plsc.bitcast(b1[r, pl.ds(g * 16, 16)], jnp.bfloat16)
                vb = plsc.bitcast(b2[r, pl.ds(g * 16, 16)], jnp.bfloat16)
                vo = jnp.maximum(va + vb, zb)
                ob[r, pl.ds(g * 16, 16)] = plsc.bitcast(vo, jnp.int32)
            return carry

        lax.fori_loop(0, CCH, body, 0)
        pltpu.sync_copy(ob, ob_hbm.at[pl.ds(tbase + c * CCH, CCH)])


@jax.jit
def kernel(x, Wg, We, be):
    x2 = x.reshape(ROWS, D)
    gate, xb, a1f, a2f, w1o, w2o = pl.pallas_call(
        _gate_kernel,
        grid=(ROWS // GTILE,),
        in_specs=[
            pl.BlockSpec((GTILE, D), lambda i: (i, 0)),
            pl.BlockSpec((E, D), lambda i: (0, 0)),
        ],
        out_specs=[
            pl.BlockSpec((GTILE, E), lambda i: (i, 0)),
            pl.BlockSpec((GTILE, D), lambda i: (i, 0)),
            pl.BlockSpec((GTILE, 1), lambda i: (i, 0)),
            pl.BlockSpec((GTILE, 1), lambda i: (i, 0)),
            pl.BlockSpec((GTILE, 1), lambda i: (i, 0)),
            pl.BlockSpec((GTILE, 1), lambda i: (i, 0)),
        ],
        out_shape=[
            jax.ShapeDtypeStruct((ROWS, E), jnp.float32),
            jax.ShapeDtypeStruct((ROWS, D), jnp.bfloat16),
            jax.ShapeDtypeStruct((ROWS, 1), jnp.float32),
            jax.ShapeDtypeStruct((ROWS, 1), jnp.float32),
            jax.ShapeDtypeStruct((ROWS, 1), jnp.float32),
            jax.ShapeDtypeStruct((ROWS, 1), jnp.float32),
        ],
    )(x2, Wg)

    ar = jnp.concatenate([a1f.reshape(ROWS), a2f.reshape(ROWS)])
    wc = jnp.concatenate([w1o.reshape(ROWS), w2o.reshape(ROWS)])

    xb_i = lax.bitcast_convert_type(
        xb.reshape(ROWS, D // 2, 2), jnp.int32)
    wr, pp, cnt, xs_i = _dispatch_kernel(ar, wc, xb_i)

    tot = cnt[:E]
    padded = ((tot + MT - 1) // MT) * MT
    ends = jnp.cumsum(padded)
    tile_expert = jnp.minimum(
        jnp.searchsorted(ends, jnp.arange(NT, dtype=jnp.int32) * MT,
                         side="right"),
        E - 1).astype(jnp.int32)


    ys_i = pl.pallas_call(
        _mm_kernel,
        grid_spec=pltpu.PrefetchScalarGridSpec(
            num_scalar_prefetch=1,
            grid=(NT,),
            in_specs=[
                pl.BlockSpec((MT, D), lambda i, s: (i, 0)),
                pl.BlockSpec((1, D, D), lambda i, s: (s[i], 0, 0)),
                pl.BlockSpec((1, 1, D), lambda i, s: (s[i], 0, 0)),
                pl.BlockSpec((1, MT, 1), lambda i, s: (i, 0, 0)),
            ],
            out_specs=pl.BlockSpec((MT, D), lambda i, s: (i, 0)),
        ),
        out_shape=jax.ShapeDtypeStruct((NPAD, D), jnp.bfloat16),
    )(tile_expert,
      lax.bitcast_convert_type(xs_i, jnp.bfloat16).reshape(NPAD, D),
      We.astype(jnp.bfloat16), be.reshape(E, 1, D), wr.reshape(NT, MT, 1))

    if False:
        y1 = jnp.take(ys_i, pp[:ROWS], axis=0)
        y2 = jnp.take(ys_i, pp[ROWS:], axis=0)
        out = jnp.maximum(y1 + y2, jnp.bfloat16(0)).astype(jnp.float32)
    else:
        ob_i = _combine_kernel(
            lax.bitcast_convert_type(ys_i.reshape(NPAD, D // 2, 2),
                                     jnp.int32), pp)
        out = lax.bitcast_convert_type(
            ob_i, jnp.bfloat16).reshape(ROWS, D).astype(jnp.float32)
    return out.reshape(B, T, D), gate.reshape(B, T, E)


# speedup vs baseline: 8.0539x; 8.0539x over previous
import functools

import jax
import jax.numpy as jnp
from jax.experimental import pallas as pl

B, T, D, E, TOP_K = 2, 2048, 1024, 8, 2
ROWS = B * T
TILE = 1024


def _moe_kernel(x_ref, wg_ref, we_ref, be_ref, out_ref, gate_ref):
    x = x_ref[...]
    xb = x.astype(jnp.bfloat16)

    wg = wg_ref[...].astype(jnp.bfloat16)
    logits = jax.lax.dot_general(
        xb, wg, (((1,), (1,)), ((), ())),
        preferred_element_type=jnp.float32)

    m = jnp.max(logits, axis=1, keepdims=True)
    eg = jnp.exp(logits - m)
    gate = eg / jnp.sum(eg, axis=1, keepdims=True)

    col = jax.lax.broadcasted_iota(jnp.int32, (TILE, E), 1)
    a1 = jnp.argmax(gate, axis=1)[:, None]
    w1 = jnp.max(gate, axis=1)[:, None]
    masked = jnp.where(col == a1, -jnp.inf, gate)
    a2 = jnp.argmax(masked, axis=1)[:, None]
    w2 = jnp.max(masked, axis=1)[:, None]

    G = jnp.where(col == a1, w1, 0.0) + jnp.where(col == a2, w2, 0.0)

    acc = jax.lax.dot_general(
        G, be_ref[...], (((1,), (0,)), ((), ())),
        preferred_element_type=jnp.float32)

    for i in range(E):
        y = jax.lax.dot_general(
            xb, we_ref[i], (((1,), (1,)), ((), ())),
            preferred_element_type=jnp.float32)
        acc = acc + G[:, i:i + 1] * y

    out_ref[...] = jnp.maximum(acc, 0.0)
    gate_ref[...] = gate


@jax.jit
def kernel(x, Wg, We, be):
    x2 = x.reshape(ROWS, D)
    we_bf16 = We.astype(jnp.bfloat16)
    grid = (ROWS // TILE,)
    out, gate = pl.pallas_call(
        _moe_kernel,
        grid=grid,
        in_specs=[
            pl.BlockSpec((TILE, D), lambda i: (i, 0)),
            pl.BlockSpec((E, D), lambda i: (0, 0)),
            pl.BlockSpec((E, D, D), lambda i: (0, 0, 0)),
            pl.BlockSpec((E, D), lambda i: (0, 0)),
        ],
        out_specs=[
            pl.BlockSpec((TILE, D), lambda i: (i, 0)),
            pl.BlockSpec((TILE, E), lambda i: (i, 0)),
        ],
        out_shape=[
            jax.ShapeDtypeStruct((ROWS, D), jnp.float32),
            jax.ShapeDtypeStruct((ROWS, E), jnp.float32),
        ],
    )(x2, Wg, we_bf16, be)
    return out.reshape(B, T, D), gate.reshape(B, T, E)
